# baseline (speedup 1.0000x reference)
import functools

import jax
import jax.numpy as jnp
from jax import lax
from jax.experimental import pallas as pl
from jax.experimental.pallas import tpu as pltpu
from jax.experimental.pallas import tpu_sc as plsc

_N_EDGES = 320000
_D = 128
_N_SEG = 10000
_NC = 2
_NS = 16
_NW = _NC * _NS
_CHUNK = 128
_N_CHUNKS = _N_EDGES // _CHUNK
_NBUF = 3
_ZROWS = 5
_SEG_PER_TILE = _N_SEG // _NS
_WB_CHUNK = 16
_N_WB_CHUNKS = _N_SEG // _WB_CHUNK


def _sc_body(x_hbm, idx2_hbm, out_hbm, acc_sh, idx_v, rows_v, zeros_v,
             *sems):
    row_sems = sems[:_NBUF]
    idx_sems = sems[_NBUF:]
    c = lax.axis_index("c")
    s = lax.axis_index("s")
    tid = c * _NS + s

    def _row_src(ch):
        off = pl.multiple_of(ch * _CHUNK, 128)
        return x_hbm.at[pl.ds(off, _CHUNK)]

    def _start_loads(ch, b):
        pltpu.async_copy(_row_src(ch), rows_v.at[b], row_sems[b])
        pltpu.async_copy(idx2_hbm.at[ch], idx_v.at[b], idx_sems[b])

    def _wait_loads(ch, b):
        pltpu.make_async_copy(_row_src(ch), rows_v.at[b], row_sems[b]).wait()
        pltpu.make_async_copy(idx2_hbm.at[ch], idx_v.at[b],
                              idx_sems[b]).wait()

    for b in range(_NBUF):
        _start_loads(tid + _NW * b, b)

    zvec = jnp.zeros((16,), jnp.float32)
    def _zero_row(i, _):
        def _zero_lane(k, _):
            zeros_v[i, pl.ds(k * 16, 16)] = zvec
            return ()
        lax.fori_loop(0, _D // 16, _zero_lane, (), unroll=True)
        return ()
    lax.fori_loop(0, _ZROWS, _zero_row, ())
    def _zero_copy(j, _):
        pltpu.sync_copy(zeros_v,
                        acc_sh.at[pl.ds(s * _SEG_PER_TILE + j * _ZROWS, _ZROWS)])
        return ()
    lax.fori_loop(0, _SEG_PER_TILE // _ZROWS, _zero_copy, ())
    plsc.subcore_barrier()

    n_tile_chunks = (_N_CHUNKS + _NW - 1) // _NW
    n_outer = (n_tile_chunks + _NBUF - 1) // _NBUF
    def _outer(jo, _):
        for b in range(_NBUF):
            ch = tid + _NW * (jo * _NBUF + b)
            @pl.when(ch < _N_CHUNKS)
            def _():
                _wait_loads(ch, b)
                pltpu.sync_copy(rows_v.at[b], acc_sh.at[idx_v.at[b]],
                                add=True)
                chn = ch + _NW * _NBUF
                @pl.when(chn < _N_CHUNKS)
                def _():
                    _start_loads(chn, b)
        return ()
    lax.fori_loop(0, n_outer, _outer, ())
    plsc.subcore_barrier()

    def _wb(j, _):
        cw = j * _NS + s
        @pl.when(cw < _N_WB_CHUNKS)
        def _():
            r0 = pl.multiple_of(cw * _WB_CHUNK, 16)
            pltpu.sync_copy(acc_sh.at[pl.ds(r0, _WB_CHUNK)],
                            out_hbm.at[c, pl.ds(r0, _WB_CHUNK)])
        return ()
    lax.fori_loop(0, (_N_WB_CHUNKS + _NS - 1) // _NS, _wb, ())


def _tc_add(a_ref, b_ref, o_ref):
    o_ref[...] = a_ref[0] + b_ref[0]


@jax.jit
def kernel(x, index):
    idx2 = index.astype(jnp.int32).reshape(_N_CHUNKS, _CHUNK)
    mesh = plsc.VectorSubcoreMesh(core_axis_name="c", subcore_axis_name="s")
    partials = pl.kernel(
        _sc_body,
        out_type=jax.ShapeDtypeStruct((_NC, _N_SEG, _D), jnp.float32),
        mesh=mesh,
        scratch_types=[
            pltpu.VMEM_SHARED((_N_SEG, _D), jnp.float32),
            pltpu.VMEM((_NBUF, _CHUNK), jnp.int32),
            pltpu.VMEM((_NBUF, _CHUNK, _D), jnp.float32),
            pltpu.VMEM((_ZROWS, _D), jnp.float32),
            *([pltpu.SemaphoreType.DMA] * (2 * _NBUF)),
        ],
    )(x, idx2)

    blk = 2000
    out = pl.pallas_call(
        _tc_add,
        grid=(_N_SEG // blk,),
        in_specs=[
            pl.BlockSpec((1, blk, _D), lambda i: (0, i, 0)),
            pl.BlockSpec((1, blk, _D), lambda i: (1, i, 0)),
        ],
        out_specs=pl.BlockSpec((blk, _D), lambda i: (i, 0)),
        out_shape=jax.ShapeDtypeStruct((_N_SEG, _D), jnp.float32),
    )(partials, partials)
    return out


# speedup vs baseline: 1.0602x; 1.0602x over previous
import functools

import jax
import jax.numpy as jnp
from jax import lax
from jax.experimental import pallas as pl
from jax.experimental.pallas import tpu as pltpu
from jax.experimental.pallas import tpu_sc as plsc

_N_EDGES = 320000
_D = 128
_N_SEG = 10000
_NC = 2
_NS = 16
_NW = _NC * _NS
_EDGES_PER_TILE = _N_EDGES // _NW
_CHUNK = 80
_N_CHUNKS = _EDGES_PER_TILE // _CHUNK
_NBUF = 4
_ZROWS = 5
_SEG_PER_TILE = _N_SEG // _NS
_WB_CHUNK = 16
_N_WB_CHUNKS = _N_SEG // _WB_CHUNK


def _sc_body(x_hbm, idx2_hbm, out_hbm, acc_sh, idx_v, rows_v, zeros_v,
             *sems):
    row_sems = sems[:_NBUF]
    idx_sems = sems[_NBUF:]
    c = lax.axis_index("c")
    s = lax.axis_index("s")
    tid = c * _NS + s
    base_chunk = tid * _N_CHUNKS

    def _row_src(ci):
        off = pl.multiple_of((base_chunk + ci) * _CHUNK, 16)
        return x_hbm.at[pl.ds(off, _CHUNK)]

    def _start_loads(ci, b):
        pltpu.async_copy(_row_src(ci), rows_v.at[b], row_sems[b])
        pltpu.async_copy(idx2_hbm.at[base_chunk + ci], idx_v.at[b],
                         idx_sems[b])

    def _wait_loads(ci, b):
        pltpu.make_async_copy(_row_src(ci), rows_v.at[b], row_sems[b]).wait()
        pltpu.make_async_copy(idx2_hbm.at[base_chunk + ci], idx_v.at[b],
                              idx_sems[b]).wait()

    for b in range(_NBUF):
        _start_loads(b, b)

    zvec = jnp.zeros((16,), jnp.float32)
    def _zero_row(i, _):
        def _zero_lane(k, _):
            zeros_v[i, pl.ds(k * 16, 16)] = zvec
            return ()
        lax.fori_loop(0, _D // 16, _zero_lane, (), unroll=True)
        return ()
    lax.fori_loop(0, _ZROWS, _zero_row, ())
    def _zero_copy(j, _):
        pltpu.sync_copy(zeros_v,
                        acc_sh.at[pl.ds(s * _SEG_PER_TILE + j * _ZROWS, _ZROWS)])
        return ()
    lax.fori_loop(0, _SEG_PER_TILE // _ZROWS, _zero_copy, ())
    plsc.subcore_barrier()

    n_outer = (_N_CHUNKS + _NBUF - 1) // _NBUF
    def _outer(jo, _):
        for b in range(_NBUF):
            ci = jo * _NBUF + b
            @pl.when(ci < _N_CHUNKS)
            def _():
                _wait_loads(ci, b)
                pltpu.sync_copy(rows_v.at[b], acc_sh.at[idx_v.at[b]],
                                add=True)
                @pl.when(ci + _NBUF < _N_CHUNKS)
                def _():
                    _start_loads(ci + _NBUF, b)
        return ()
    lax.fori_loop(0, n_outer, _outer, ())
    plsc.subcore_barrier()

    def _wb(j, _):
        cw = j * _NS + s
        @pl.when(cw < _N_WB_CHUNKS)
        def _():
            r0 = pl.multiple_of(cw * _WB_CHUNK, 16)
            pltpu.sync_copy(acc_sh.at[pl.ds(r0, _WB_CHUNK)],
                            out_hbm.at[c, pl.ds(r0, _WB_CHUNK)])
        return ()
    lax.fori_loop(0, (_N_WB_CHUNKS + _NS - 1) // _NS, _wb, ())


def _tc_add(a_ref, b_ref, o_ref):
    o_ref[...] = a_ref[0] + b_ref[0]


@jax.jit
def kernel(x, index):
    idx2 = index.astype(jnp.int32).reshape(_NW * _N_CHUNKS, _CHUNK)
    mesh = plsc.VectorSubcoreMesh(core_axis_name="c", subcore_axis_name="s")
    partials = pl.kernel(
        _sc_body,
        out_type=jax.ShapeDtypeStruct((_NC, _N_SEG, _D), jnp.float32),
        mesh=mesh,
        scratch_types=[
            pltpu.VMEM_SHARED((_N_SEG, _D), jnp.float32),
            pltpu.VMEM((_NBUF, _CHUNK), jnp.int32),
            pltpu.VMEM((_NBUF, _CHUNK, _D), jnp.float32),
            pltpu.VMEM((_ZROWS, _D), jnp.float32),
            *([pltpu.SemaphoreType.DMA] * (2 * _NBUF)),
        ],
    )(x, idx2)

    blk = 2000
    out = pl.pallas_call(
        _tc_add,
        grid=(_N_SEG // blk,),
        in_specs=[
            pl.BlockSpec((1, blk, _D), lambda i: (0, i, 0)),
            pl.BlockSpec((1, blk, _D), lambda i: (1, i, 0)),
        ],
        out_specs=pl.BlockSpec((blk, _D), lambda i: (i, 0)),
        out_shape=jax.ShapeDtypeStruct((_N_SEG, _D), jnp.float32),
    )(partials, partials)
    return out


# speedup vs baseline: 1.0950x; 1.0328x over previous
import functools

import jax
import jax.numpy as jnp
from jax import lax
from jax.experimental import pallas as pl
from jax.experimental.pallas import tpu as pltpu
from jax.experimental.pallas import tpu_sc as plsc

_N_EDGES = 320000
_D = 128
_N_SEG = 10000
_NC = 2
_NS = 16
_NW = _NC * _NS
_EDGES_PER_TILE = _N_EDGES // _NW
_CHUNK = 128
_NFULL = _EDGES_PER_TILE // _CHUNK
_TAIL = _EDGES_PER_TILE - _NFULL * _CHUNK
_NBUF = 3
_SEG_PER_TILE = _N_SEG // _NS
_WB_CHUNK = 16
_N_WB_CHUNKS = _N_SEG // _WB_CHUNK


def _sc_body(x_hbm, idx_hbm, out_hbm, acc_sh, idx_v, idx_t, rows_v, *sems):
    row_sems = sems[:_NBUF]
    idx_sems = sems[_NBUF:2 * _NBUF]
    tail_sem = sems[2 * _NBUF]
    c = lax.axis_index("c")
    s = lax.axis_index("s")
    tid = c * _NS + s
    base = tid * _EDGES_PER_TILE

    def _row_src(ci):
        off = pl.multiple_of(base + ci * _CHUNK, 16)
        return x_hbm.at[pl.ds(off, _CHUNK)]

    def _idx_src(ci):
        off = pl.multiple_of(base + ci * _CHUNK, 16)
        return idx_hbm.at[pl.ds(off, _CHUNK)]

    def _start_loads(ci, b):
        pltpu.async_copy(_row_src(ci), rows_v.at[b], row_sems[b])
        pltpu.async_copy(_idx_src(ci), idx_v.at[b], idx_sems[b])

    def _wait_loads(ci, b):
        pltpu.make_async_copy(_row_src(ci), rows_v.at[b], row_sems[b]).wait()
        pltpu.make_async_copy(_idx_src(ci), idx_v.at[b],
                              idx_sems[b]).wait()

    for b in range(_NBUF - 1):
        _start_loads(b, b)

    zb = _NBUF - 1
    zvec = jnp.zeros((16,), jnp.float32)
    def _zero_row(i, _):
        def _zero_lane(k, _):
            rows_v[zb, i, pl.ds(k * 16, 16)] = zvec
            return ()
        lax.fori_loop(0, _D // 16, _zero_lane, (), unroll=True)
        return ()
    lax.fori_loop(0, _CHUNK, _zero_row, ())
    seg0 = s * _SEG_PER_TILE
    for j in range(_SEG_PER_TILE // _CHUNK):
        pltpu.sync_copy(rows_v.at[zb],
                        acc_sh.at[pl.ds(seg0 + j * _CHUNK, _CHUNK)])
    rem = _SEG_PER_TILE % _CHUNK
    pltpu.sync_copy(rows_v.at[zb, pl.ds(0, rem)],
                    acc_sh.at[pl.ds(seg0 + _SEG_PER_TILE - rem, rem)])
    _start_loads(zb, zb)
    plsc.subcore_barrier()

    def _outer(jo, _):
        for b in range(_NBUF):
            ci = jo * _NBUF + b
            _wait_loads(ci, b)
            pltpu.sync_copy(rows_v.at[b], acc_sh.at[idx_v.at[b]], add=True)
            _start_loads(ci + _NBUF, b)
        return ()
    lax.fori_loop(0, _NFULL // _NBUF - 1, _outer, ())
    for b in range(_NBUF):
        ci = _NFULL - _NBUF + b
        _wait_loads(ci, b)
        pltpu.sync_copy(rows_v.at[b], acc_sh.at[idx_v.at[b]], add=True)

    tail_off = pl.multiple_of(base + _NFULL * _CHUNK, 16)
    pltpu.sync_copy(idx_hbm.at[pl.ds(tail_off, _TAIL)], idx_t)
    pltpu.async_copy(x_hbm.at[pl.ds(tail_off, _TAIL)],
                     rows_v.at[0, pl.ds(0, _TAIL)], tail_sem)
    pltpu.make_async_copy(x_hbm.at[pl.ds(tail_off, _TAIL)],
                          rows_v.at[0, pl.ds(0, _TAIL)], tail_sem).wait()
    pltpu.sync_copy(rows_v.at[0, pl.ds(0, _TAIL)], acc_sh.at[idx_t],
                    add=True)
    plsc.subcore_barrier()

    def _wb(j, _):
        cw = j * _NS + s
        r0 = pl.multiple_of(cw * _WB_CHUNK, 16)
        pltpu.sync_copy(acc_sh.at[pl.ds(r0, _WB_CHUNK)],
                        out_hbm.at[c, pl.ds(r0, _WB_CHUNK)])
        return ()
    lax.fori_loop(0, _N_WB_CHUNKS // _NS, _wb, ())
    @pl.when(s == 0)
    def _():
        r0 = (_N_WB_CHUNKS // _NS) * _NS * _WB_CHUNK
        pltpu.sync_copy(acc_sh.at[pl.ds(r0, _WB_CHUNK)],
                        out_hbm.at[c, pl.ds(r0, _WB_CHUNK)])


def _tc_add(a_ref, b_ref, o_ref):
    o_ref[...] = a_ref[0] + b_ref[0]


@jax.jit
def kernel(x, index):
    idx = index.astype(jnp.int32)
    mesh = plsc.VectorSubcoreMesh(core_axis_name="c", subcore_axis_name="s")
    partials = pl.kernel(
        _sc_body,
        out_type=jax.ShapeDtypeStruct((_NC, _N_SEG, _D), jnp.float32),
        mesh=mesh,
        scratch_types=[
            pltpu.VMEM_SHARED((_N_SEG, _D), jnp.float32),
            pltpu.VMEM((_NBUF, _CHUNK), jnp.int32),
            pltpu.VMEM((_TAIL,), jnp.int32),
            pltpu.VMEM((_NBUF, _CHUNK, _D), jnp.float32),
            *([pltpu.SemaphoreType.DMA] * (2 * _NBUF + 1)),
        ],
    )(x, idx)

    blk = 2000
    out = pl.pallas_call(
        _tc_add,
        grid=(_N_SEG // blk,),
        in_specs=[
            pl.BlockSpec((1, blk, _D), lambda i: (0, i, 0)),
            pl.BlockSpec((1, blk, _D), lambda i: (1, i, 0)),
        ],
        out_specs=pl.BlockSpec((blk, _D), lambda i: (i, 0)),
        out_shape=jax.ShapeDtypeStruct((_N_SEG, _D), jnp.float32),
    )(partials, partials)
    return out
